# baseline (speedup 1.0000x reference)
import functools

import jax
import jax.numpy as jnp
from jax import lax
from jax.experimental import pallas as pl
from jax.experimental.pallas import tpu as pltpu
from jax.experimental.pallas import tpu_sc as plsc

_SEQ = 200
_D = 128
_NPART = 13


def _sc_body(x0_hbm, emb1_hbm, out_hbm, idx_v, rows_v, sem):
    w = lax.axis_index("s")

    @pl.when(w < _NPART)
    def _gather():
        base = jnp.minimum(w * 16, _SEQ - 16)
        pltpu.sync_copy(x0_hbm.at[pl.ds(base, 16)], idx_v)
        pltpu.async_copy(emb1_hbm.at[idx_v], rows_v, sem).wait()
        pltpu.sync_copy(rows_v, out_hbm.at[pl.ds(w * 16, 16)])


_sc_gather = functools.partial(
    pl.kernel,
    _sc_body,
    out_type=jax.ShapeDtypeStruct((_NPART * 16, _D), jnp.float32),
    scratch_types=[
        pltpu.VMEM((16,), jnp.int32),
        pltpu.VMEM((16, _D), jnp.float32),
        pltpu.SemaphoreType.DMA,
    ],
    mesh=plsc.VectorSubcoreMesh(core_axis_name="c", subcore_axis_name="s",
                                num_cores=1),
)()


def _tc_dense_body(x1_ref, rows_ref, w1_ref, b1_ref, v_ref, out_ref):
    total = jnp.sum(rows_ref[...], axis=0, keepdims=True)
    dup = jnp.sum(rows_ref[192:200, :], axis=0, keepdims=True)
    m = (total - dup) * (1.0 / _SEQ)
    y1 = lax.dot_general(m, w1_ref[...], (((1,), (1,)), ((), ())),
                         preferred_element_type=jnp.float32)
    y1 = y1 + b1_ref[...]
    s = jnp.sum(y1 * v_ref[0])
    out_ref[...] = jax.nn.sigmoid(s) * jnp.ones((1, _D), jnp.float32)


def kernel(x0, x1, emb1, W1, b1, emb2):
    rows = _sc_gather(x0, emb1)
    grid_spec = pltpu.PrefetchScalarGridSpec(
        num_scalar_prefetch=1,
        grid=(1,),
        in_specs=[
            pl.BlockSpec((_NPART * 16, _D), lambda i, x1r: (0, 0)),
            pl.BlockSpec((_D, _D), lambda i, x1r: (0, 0)),
            pl.BlockSpec((1, _D), lambda i, x1r: (0, 0)),
            pl.BlockSpec((1, 1, _D), lambda i, x1r: (x1r[0], 0, 0)),
        ],
        out_specs=pl.BlockSpec((1, _D), lambda i, x1r: (0, 0)),
    )
    out = pl.pallas_call(
        _tc_dense_body,
        grid_spec=grid_spec,
        out_shape=jax.ShapeDtypeStruct((1, _D), jnp.float32),
    )(x1, rows, W1, b1.reshape(1, _D), emb2.reshape(-1, 1, _D))
    return out[0, 0]


# speedup vs baseline: 1.0171x; 1.0171x over previous
import functools

import jax
import jax.numpy as jnp
from jax import lax
from jax.experimental import pallas as pl
from jax.experimental.pallas import tpu as pltpu
from jax.experimental.pallas import tpu_sc as plsc

_SEQ = 200
_D = 128
_NCHUNK = _D // 16
_NPART = 13


def _sc_body(x0_hbm, x1_hbm, emb1_hbm, emb2_hbm, w1_hbm, b1_hbm, out_hbm,
             idx_v, rows_v, acc_v, idx2_v, row2_v, local_v, w1_v, b1_v,
             cv_v, cl_v, stage_s, cstage_s, sem, sem_w1, sem_b1):
    w = lax.axis_index("s")

    w1_dma = pltpu.make_async_copy(w1_hbm.at[pl.ds(w * 8, 8)], w1_v, sem_w1)
    w1_dma.start()
    b1_dma = pltpu.make_async_copy(b1_hbm, b1_v, sem_b1)

    @pl.when(w == 15)
    def _start_b1():
        b1_dma.start()

    @pl.when(w < _NPART)
    def _gather():
        base = jnp.minimum(w * 16, _SEQ - 16)
        lo = w * 16 - base
        pltpu.sync_copy(x0_hbm.at[pl.ds(base, 16)], idx_v)
        pltpu.async_copy(emb1_hbm.at[idx_v], rows_v, sem).wait()

        def body(r, carry):
            wt = jnp.where(r >= lo, 1.0, 0.0)
            return tuple(carry[ci] + wt * rows_v[r, pl.ds(ci * 16, 16)]
                         for ci in range(_NCHUNK))

        zero = jnp.zeros((16,), jnp.float32)
        acc = lax.fori_loop(0, 16, body, (zero,) * _NCHUNK)
        for ci in range(_NCHUNK):
            acc_v[pl.ds(ci * 16, 16)] = acc[ci]
        pltpu.sync_copy(acc_v, stage_s.at[w])

    @pl.when(w == _NPART)
    def _gather_emb2():
        pltpu.sync_copy(x1_hbm, idx2_v)
        pltpu.async_copy(emb2_hbm.at[idx2_v], row2_v, sem).wait()
        pltpu.sync_copy(row2_v, stage_s.at[pl.ds(_NPART, 1)])

    plsc.subcore_barrier()

    pltpu.sync_copy(stage_s, local_v)
    inv = 1.0 / _SEQ
    m = []
    v = []
    for ci in range(_NCHUNK):
        s = local_v[0, pl.ds(ci * 16, 16)]
        for p in range(1, _NPART):
            s = s + local_v[p, pl.ds(ci * 16, 16)]
        m.append(s * inv)
        v.append(local_v[_NPART, pl.ds(ci * 16, 16)])

    w1_dma.wait()

    c0 = w // 2
    off = (w % 2) * 8
    vchunk = jnp.zeros((16,), jnp.float32)
    for ci in range(_NCHUNK):
        vchunk = jnp.where(ci == c0, v[ci], vchunk)

    def mv_body(r, t):
        bidx = jnp.full((16,), off + r, jnp.int32)
        bv = vchunk.at[bidx].get(mode="promise_in_bounds")
        return tuple(t[ci] + bv * w1_v[r, pl.ds(ci * 16, 16)]
                     for ci in range(_NCHUNK))

    zero = jnp.zeros((16,), jnp.float32)
    t = lax.fori_loop(0, 8, mv_body, (zero,) * _NCHUNK)
    pvec = t[0] * m[0]
    for ci in range(1, _NCHUNK):
        pvec = pvec + t[ci] * m[ci]

    @pl.when(w == 15)
    def _bias_term():
        b1_dma.wait()
        bias = b1_v[pl.ds(0, 16)] * v[0]
        for ci in range(1, _NCHUNK):
            bias = bias + b1_v[pl.ds(ci * 16, 16)] * v[ci]
        cv_v[...] = pvec + bias

    @pl.when(w < 15)
    def _no_bias():
        cv_v[...] = pvec

    pltpu.sync_copy(cv_v, cstage_s.at[w])
    plsc.subcore_barrier()

    @pl.when(w == 0)
    def _finish():
        pltpu.sync_copy(cstage_s, cl_v)
        tot = cl_v[0, pl.ds(0, 16)]
        for p in range(1, 16):
            tot = tot + cl_v[p, pl.ds(0, 16)]
        lane = lax.iota(jnp.int32, 16)
        for step in (1, 2, 4, 8):
            ridx = (lane + step) & 15
            tot = tot + tot.at[ridx].get(mode="promise_in_bounds")
        y = 1.0 / (1.0 + jnp.exp(-tot))
        cv_v[...] = y
        pltpu.sync_copy(cv_v, out_hbm)


_sc_all = functools.partial(
    pl.kernel,
    _sc_body,
    out_type=jax.ShapeDtypeStruct((16,), jnp.float32),
    scratch_types=[
        pltpu.VMEM((16,), jnp.int32),
        pltpu.VMEM((16, _D), jnp.float32),
        pltpu.VMEM((_D,), jnp.float32),
        pltpu.VMEM((1,), jnp.int32),
        pltpu.VMEM((1, _D), jnp.float32),
        pltpu.VMEM((_NPART + 1, _D), jnp.float32),
        pltpu.VMEM((8, _D), jnp.float32),
        pltpu.VMEM((_D,), jnp.float32),
        pltpu.VMEM((16,), jnp.float32),
        pltpu.VMEM((16, 16), jnp.float32),
        pltpu.VMEM_SHARED((_NPART + 1, _D), jnp.float32),
        pltpu.VMEM_SHARED((16, 16), jnp.float32),
        pltpu.SemaphoreType.DMA,
        pltpu.SemaphoreType.DMA,
        pltpu.SemaphoreType.DMA,
    ],
    mesh=plsc.VectorSubcoreMesh(core_axis_name="c", subcore_axis_name="s",
                                num_cores=1),
)()


def kernel(x0, x1, emb1, W1, b1, emb2):
    out = _sc_all(x0, x1, emb1, emb2, W1, b1)
    return out[0]


# speedup vs baseline: 1.0265x; 1.0092x over previous
import functools

import jax
import jax.numpy as jnp
from jax import lax
from jax.experimental import pallas as pl
from jax.experimental.pallas import tpu as pltpu
from jax.experimental.pallas import tpu_sc as plsc

_SEQ = 200
_D = 128
_NCHUNK = _D // 16
_NPART = 13


def _sc_body(x0_hbm, x1_hbm, emb1_hbm, emb2_hbm, w1_hbm, b1_hbm, out_hbm,
             idx_v, rows_v, acc_v, idx2_v, vrow_v, big_v, w1_v, b1_v,
             cv_v, stage_s, tstage_s, sem, sem_w1, sem_b1):
    w = lax.axis_index("s")

    w1_dma = pltpu.make_async_copy(w1_hbm.at[pl.ds(w * 8, 8)], w1_v, sem_w1)
    w1_dma.start()
    b1_dma = pltpu.make_async_copy(b1_hbm, b1_v, sem_b1)

    @pl.when(w == 0)
    def _start_b1():
        b1_dma.start()

    @pl.when(w < _NPART)
    def _gather():
        base = jnp.minimum(w * 16, _SEQ - 16)
        lo = w * 16 - base
        pltpu.sync_copy(x0_hbm.at[pl.ds(base, 16)], idx_v)
        pltpu.async_copy(emb1_hbm.at[idx_v], rows_v, sem).wait()

        def body(r, carry):
            wt = jnp.where(r >= lo, 1.0, 0.0)
            return tuple(carry[ci] + wt * rows_v[r, pl.ds(ci * 16, 16)]
                         for ci in range(_NCHUNK))

        zero = jnp.zeros((16,), jnp.float32)
        acc = lax.fori_loop(0, 16, body, (zero,) * _NCHUNK)
        for ci in range(_NCHUNK):
            acc_v[pl.ds(ci * 16, 16)] = acc[ci]
        pltpu.sync_copy(acc_v, stage_s.at[w])

    @pl.when(w == _NPART)
    def _gather_emb2():
        pltpu.sync_copy(x1_hbm, idx2_v)
        pltpu.async_copy(emb2_hbm.at[idx2_v],
                         vrow_v.at[pl.ds(0, 1)], sem).wait()
        pltpu.sync_copy(vrow_v.at[pl.ds(0, 1)], stage_s.at[pl.ds(_NPART, 1)])

    plsc.subcore_barrier()

    pltpu.sync_copy(stage_s.at[pl.ds(_NPART, 1)], vrow_v)
    c0 = w // 2
    off = (w % 2) * 8
    vchunk = vrow_v[0, pl.ds(0, 16)]
    for ci in range(1, _NCHUNK):
        vc = vrow_v[0, pl.ds(ci * 16, 16)]
        vchunk = jnp.where(ci == c0, vc, vchunk)

    w1_dma.wait()

    def mv_body(r, t):
        bidx = jnp.full((16,), off + r, jnp.int32)
        bv = vchunk.at[bidx].get(mode="promise_in_bounds")
        return tuple(t[ci] + bv * w1_v[r, pl.ds(ci * 16, 16)]
                     for ci in range(_NCHUNK))

    zero = jnp.zeros((16,), jnp.float32)
    t = lax.fori_loop(0, 8, mv_body, (zero,) * _NCHUNK)
    for ci in range(_NCHUNK):
        acc_v[pl.ds(ci * 16, 16)] = t[ci]
    pltpu.sync_copy(acc_v, tstage_s.at[w])
    plsc.subcore_barrier()

    @pl.when(w == 0)
    def _finish():
        pltpu.sync_copy(stage_s, big_v.at[pl.ds(0, _NPART + 1)])
        pltpu.sync_copy(tstage_s, big_v.at[pl.ds(_NPART + 1, 16)])

        def row_sum(lo, hi):
            def body(p, carry):
                return tuple(carry[ci] + big_v[p, pl.ds(ci * 16, 16)]
                             for ci in range(_NCHUNK))
            init = tuple(big_v[lo, pl.ds(ci * 16, 16)]
                         for ci in range(_NCHUNK))
            return lax.fori_loop(lo + 1, hi, body, init)

        part = row_sum(0, _NPART)
        tt = row_sum(_NPART + 1, _NPART + 1 + 16)
        b1_dma.wait()
        inv = 1.0 / _SEQ
        pvec = jnp.zeros((16,), jnp.float32)
        for ci in range(_NCHUNK):
            vc = big_v[_NPART, pl.ds(ci * 16, 16)]
            pvec = pvec + (part[ci] * inv) * tt[ci] \
                + b1_v[pl.ds(ci * 16, 16)] * vc
        lane = lax.iota(jnp.int32, 16)
        for step in (1, 2, 4, 8):
            ridx = (lane + step) & 15
            pvec = pvec + pvec.at[ridx].get(mode="promise_in_bounds")
        y = 1.0 / (1.0 + jnp.exp(-pvec))
        cv_v[...] = y
        pltpu.sync_copy(cv_v, out_hbm)


_sc_all = functools.partial(
    pl.kernel,
    _sc_body,
    out_type=jax.ShapeDtypeStruct((16,), jnp.float32),
    scratch_types=[
        pltpu.VMEM((16,), jnp.int32),
        pltpu.VMEM((16, _D), jnp.float32),
        pltpu.VMEM((_D,), jnp.float32),
        pltpu.VMEM((1,), jnp.int32),
        pltpu.VMEM((1, _D), jnp.float32),
        pltpu.VMEM((_NPART + 1 + 16, _D), jnp.float32),
        pltpu.VMEM((8, _D), jnp.float32),
        pltpu.VMEM((_D,), jnp.float32),
        pltpu.VMEM((16,), jnp.float32),
        pltpu.VMEM_SHARED((_NPART + 1, _D), jnp.float32),
        pltpu.VMEM_SHARED((16, _D), jnp.float32),
        pltpu.SemaphoreType.DMA,
        pltpu.SemaphoreType.DMA,
        pltpu.SemaphoreType.DMA,
    ],
    mesh=plsc.VectorSubcoreMesh(core_axis_name="c", subcore_axis_name="s",
                                num_cores=1),
)()


def kernel(x0, x1, emb1, W1, b1, emb2):
    out = _sc_all(x0, x1, emb1, emb2, W1, b1)
    return out[0]
